# baseline (speedup 1.0000x reference)
import functools

import jax
import jax.numpy as jnp
from jax import lax
from jax.experimental import pallas as pl
from jax.experimental.pallas import tpu as pltpu
from jax.experimental.pallas import tpu_sc as plsc

NN = 10000
NE = 320000
DF = 128
NH = 64
NCLS = 16

NC = 2
NS = 16
NW = NC * NS
EPW = NE // NW
CHUNK = 80
NITER = EPW // CHUNK
NPAD = 10240
RPT = NPAD // NS
LAG = 8

_SC_MESH = plsc.VectorSubcoreMesh(core_axis_name="c", subcore_axis_name="s")
_SC_PARAMS = pltpu.CompilerParams(use_tc_tiling_on_sc=False)


def _edge_ring(tab, acc_sh, idxs_all, idxd_all, rows, gsems, ssems):
    for b in range(3):
        pltpu.async_copy(tab.at[idxs_all.at[b]], rows[b], gsems[b])

    def body(i, carry):
        for u in range(4):
            j = 4 * i + u
            pltpu.make_async_copy(tab.at[idxs_all.at[j]], rows[u],
                                  gsems[u]).wait()
            pltpu.async_copy(rows[u], acc_sh.at[idxd_all.at[j]], ssems[u],
                             add=True)

            @pl.when(j >= 1)
            def _ws():
                pltpu.make_async_copy(rows[(u - 1) % 4],
                                      acc_sh.at[idxd_all.at[0]],
                                      ssems[(u - 1) % 4]).wait()

            @pl.when(j + 3 < NITER)
            def _g():
                pltpu.async_copy(tab.at[idxs_all.at[j + 3]],
                                 rows[(u + 3) % 4], gsems[(u + 3) % 4])

        return carry

    nfull = NITER // 4
    lax.fori_loop(0, nfull, body, 0)
    for j in range(nfull * 4, NITER):
        u = j % 4
        pltpu.make_async_copy(tab.at[idxs_all.at[j]], rows[u],
                              gsems[u]).wait()
        pltpu.async_copy(rows[u], acc_sh.at[idxd_all.at[j]], ssems[u],
                         add=True)
        pltpu.make_async_copy(rows[(u - 1) % 4], acc_sh.at[idxd_all.at[0]],
                              ssems[(u - 1) % 4]).wait()
    last = (NITER - 1) % 4
    pltpu.make_async_copy(rows[last], acc_sh.at[idxd_all.at[0]],
                          ssems[last]).wait()


def _deg_kernel(ei_hbm, dego_hbm, degi_hbm, idxs_all, idxd_all,
                ones_v, zbuf, dego_sh, degi_sh, isem, ssem):
    c = lax.axis_index("c")
    s = lax.axis_index("s")
    wid = c * NS + s

    cps = pltpu.async_copy(ei_hbm.at[0, wid], idxs_all, isem)
    cpd = pltpu.async_copy(ei_hbm.at[1, wid], idxd_all, isem)

    def fill(i, carry):
        ones_v[pl.ds(i * 16, 16)] = jnp.ones((16,), jnp.float32)
        return carry

    lax.fori_loop(0, 7, fill, 0)

    def zfill(i, carry):
        zbuf[pl.ds(i * 16, 16)] = jnp.zeros((16,), jnp.float32)
        return carry

    lax.fori_loop(0, RPT // 16, zfill, 0)
    pltpu.sync_copy(zbuf, dego_sh.at[pl.ds(s * RPT, RPT)])
    pltpu.sync_copy(zbuf, degi_sh.at[pl.ds(s * RPT, RPT)])
    cps.wait()
    cpd.wait()
    plsc.subcore_barrier()

    ones_c = ones_v.at[pl.ds(0, CHUNK)]

    def body(j, carry):
        @pl.when(j >= LAG)
        def _drain():
            pltpu.make_async_copy(ones_c, dego_sh.at[idxs_all.at[0]],
                                  ssem).wait()
            pltpu.make_async_copy(ones_c, degi_sh.at[idxd_all.at[0]],
                                  ssem).wait()

        pltpu.async_copy(ones_c, dego_sh.at[idxs_all.at[j]], ssem, add=True)
        pltpu.async_copy(ones_c, degi_sh.at[idxd_all.at[j]], ssem, add=True)
        return carry

    lax.fori_loop(0, NITER, body, 0)

    def drain(j, carry):
        pltpu.make_async_copy(ones_c, dego_sh.at[idxs_all.at[0]], ssem).wait()
        pltpu.make_async_copy(ones_c, degi_sh.at[idxd_all.at[0]], ssem).wait()
        return carry

    lax.fori_loop(0, LAG, drain, 0)
    plsc.subcore_barrier()

    pltpu.sync_copy(dego_sh.at[pl.ds(s * RPT, RPT)],
                    dego_hbm.at[c, 0, pl.ds(s * RPT, RPT)])
    pltpu.sync_copy(degi_sh.at[pl.ds(s * RPT, RPT)],
                    degi_hbm.at[c, 0, pl.ds(s * RPT, RPT)])


_deg_call = pl.kernel(
    _deg_kernel,
    out_type=[jax.ShapeDtypeStruct((NC, 1, NPAD), jnp.float32),
              jax.ShapeDtypeStruct((NC, 1, NPAD), jnp.float32)],
    mesh=_SC_MESH,
    compiler_params=_SC_PARAMS,
    scratch_types=[
        pltpu.VMEM((NITER, CHUNK), jnp.int32),
        pltpu.VMEM((NITER, CHUNK), jnp.int32),
        pltpu.VMEM((112,), jnp.float32),
        pltpu.VMEM((RPT,), jnp.float32),
        pltpu.VMEM_SHARED((NPAD,), jnp.float32),
        pltpu.VMEM_SHARED((NPAD,), jnp.float32),
        pltpu.SemaphoreType.DMA,
        pltpu.SemaphoreType.DMA,
    ],
)


def _edge_kernel(F, ei_hbm, tab_hbm, acc_hbm,
                 idxs_all, idxd_all, r0, r1, r2, r3, zbuf, acc_sh, tab_sh,
                 isem, g0, g1, g2, g3, s0, s1, s2, s3):
    c = lax.axis_index("c")
    s = lax.axis_index("s")
    wid = c * NS + s
    rows = [r0, r1, r2, r3]
    gsems = [g0, g1, g2, g3]
    ssems = [s0, s1, s2, s3]

    cps = pltpu.async_copy(ei_hbm.at[0, wid], idxs_all, isem)
    cpd = pltpu.async_copy(ei_hbm.at[1, wid], idxd_all, isem)
    cpt = pltpu.async_copy(tab_hbm.at[pl.ds(s * RPT, RPT)],
                           tab_sh.at[pl.ds(s * RPT, RPT)], isem)

    zrows = 32
    z16 = jnp.zeros((16,), jnp.float32)

    def zero_row(i, carry):
        for jj in range(F // 16):
            zbuf[i, pl.ds(jj * 16, 16)] = z16
        return carry

    lax.fori_loop(0, zrows, zero_row, 0)

    def zero_slice(k, carry):
        pltpu.sync_copy(zbuf, acc_sh.at[pl.ds(s * RPT + k * zrows, zrows)])
        return carry

    lax.fori_loop(0, RPT // zrows, zero_slice, 0)
    cps.wait()
    cpd.wait()
    cpt.wait()
    plsc.subcore_barrier()

    _edge_ring(tab_sh, acc_sh, idxs_all, idxd_all, rows, gsems, ssems)
    plsc.subcore_barrier()

    pltpu.sync_copy(acc_sh.at[pl.ds(s * RPT, RPT)],
                    acc_hbm.at[c, pl.ds(s * RPT, RPT)])


def _make_edge_call(F):
    return pl.kernel(
        functools.partial(_edge_kernel, F),
        out_type=jax.ShapeDtypeStruct((NC, NPAD, F), jnp.float32),
        mesh=_SC_MESH,
        compiler_params=_SC_PARAMS,
        scratch_types=[
            pltpu.VMEM((NITER, CHUNK), jnp.int32),
            pltpu.VMEM((NITER, CHUNK), jnp.int32),
            pltpu.VMEM((CHUNK, F), jnp.float32),
            pltpu.VMEM((CHUNK, F), jnp.float32),
            pltpu.VMEM((CHUNK, F), jnp.float32),
            pltpu.VMEM((CHUNK, F), jnp.float32),
            pltpu.VMEM((32, F), jnp.float32),
            pltpu.VMEM_SHARED((NPAD, F), jnp.float32),
            pltpu.VMEM_SHARED((NPAD, F), jnp.float32),
        ] + [pltpu.SemaphoreType.DMA] * 9,
    )


_edge_call_h = _make_edge_call(NH)
_edge_call_c = _make_edge_call(NCLS)


BN = 1280


def _mm_scale_body(x_ref, w_ref, norm_ref, o_ref):
    z = jnp.dot(x_ref[...], w_ref[...], preferred_element_type=jnp.float32)
    o_ref[...] = z * norm_ref[...]


_mm_scale = pl.pallas_call(
    _mm_scale_body,
    grid=(NPAD // BN,),
    in_specs=[
        pl.BlockSpec((BN, DF), lambda i: (i, 0)),
        pl.BlockSpec((DF, NH), lambda i: (0, 0)),
        pl.BlockSpec((BN, 1), lambda i: (i, 0)),
    ],
    out_specs=pl.BlockSpec((BN, NH), lambda i: (i, 0)),
    out_shape=jax.ShapeDtypeStruct((NPAD, NH), jnp.float32),
)


def _mid_body(acc_ref, ni_ref, no_ref, b1_ref, w2_ref, o_ref):
    h = acc_ref[0] + acc_ref[1]
    h = jnp.maximum(h * ni_ref[...] + b1_ref[...], 0.0)
    o_ref[...] = jnp.dot(h * no_ref[...], w2_ref[...],
                         preferred_element_type=jnp.float32)


_mid = pl.pallas_call(
    _mid_body,
    grid=(NPAD // BN,),
    in_specs=[
        pl.BlockSpec((NC, BN, NH), lambda i: (0, i, 0)),
        pl.BlockSpec((BN, 1), lambda i: (i, 0)),
        pl.BlockSpec((BN, 1), lambda i: (i, 0)),
        pl.BlockSpec((1, NH), lambda i: (0, 0)),
        pl.BlockSpec((NH, NCLS), lambda i: (0, 0)),
    ],
    out_specs=pl.BlockSpec((BN, NCLS), lambda i: (i, 0)),
    out_shape=jax.ShapeDtypeStruct((NPAD, NCLS), jnp.float32),
)


def _fin_body(acc_ref, ni_ref, b2_ref, o_ref):
    o_ref[...] = (acc_ref[0] + acc_ref[1]) * ni_ref[...] + b2_ref[...]


_fin = pl.pallas_call(
    _fin_body,
    grid=(NPAD // BN,),
    in_specs=[
        pl.BlockSpec((NC, BN, NCLS), lambda i: (0, i, 0)),
        pl.BlockSpec((BN, 1), lambda i: (i, 0)),
        pl.BlockSpec((1, NCLS), lambda i: (0, 0)),
    ],
    out_specs=pl.BlockSpec((BN, NCLS), lambda i: (i, 0)),
    out_shape=jax.ShapeDtypeStruct((NN, NCLS), jnp.float32),
)


@jax.jit
def kernel(features, edge_index, W1, b1, W2, b2):
    ei = edge_index.reshape(2, NW, NITER, CHUNK)

    dego, degi = _deg_call(ei)
    deg_out = dego[0, 0] + dego[1, 0]
    deg_in = degi[0, 0] + degi[1, 0]
    norm_out = lax.rsqrt(jnp.maximum(deg_out, 1.0)).reshape(NPAD, 1)
    norm_in = lax.rsqrt(jnp.maximum(deg_in, 1.0)).reshape(NPAD, 1)

    hw1 = _mm_scale(features, W1, norm_out)
    acc1 = _edge_call_h(ei, hw1)
    hw2 = _mid(acc1, norm_in, norm_out, b1.reshape(1, NH), W2)
    acc2 = _edge_call_c(ei, hw2)
    return _fin(acc2, norm_in, b2.reshape(1, NCLS))


# speedup vs baseline: 1.0217x; 1.0217x over previous
import functools

import jax
import jax.numpy as jnp
from jax import lax
from jax.experimental import pallas as pl
from jax.experimental.pallas import tpu as pltpu
from jax.experimental.pallas import tpu_sc as plsc

NN = 10000
NE = 320000
DF = 128
NH = 64
NCLS = 16

NC = 2
NS = 16
NW = NC * NS
EPW = NE // NW
CHUNK = 80
NITER = EPW // CHUNK
NPAD = 10240
RPT = NPAD // NS
LAG = 8

_SC_MESH = plsc.VectorSubcoreMesh(core_axis_name="c", subcore_axis_name="s")
_SC_PARAMS = pltpu.CompilerParams(use_tc_tiling_on_sc=False,
                                  needs_layout_passes=False)


def _edge_ring(tab, acc_sh, idxs_all, idxd_all, rows, gsems, ssems):
    for b in range(3):
        pltpu.async_copy(tab.at[idxs_all.at[b]], rows[b], gsems[b])

    def body(i, carry):
        for u in range(4):
            j = 4 * i + u
            pltpu.make_async_copy(tab.at[idxs_all.at[j]], rows[u],
                                  gsems[u]).wait()
            pltpu.async_copy(rows[u], acc_sh.at[idxd_all.at[j]], ssems[u],
                             add=True)

            @pl.when(j >= 1)
            def _ws():
                pltpu.make_async_copy(rows[(u - 1) % 4],
                                      acc_sh.at[idxd_all.at[0]],
                                      ssems[(u - 1) % 4]).wait()

            @pl.when(j + 3 < NITER)
            def _g():
                pltpu.async_copy(tab.at[idxs_all.at[j + 3]],
                                 rows[(u + 3) % 4], gsems[(u + 3) % 4])

        return carry

    nfull = NITER // 4
    lax.fori_loop(0, nfull, body, 0)
    for j in range(nfull * 4, NITER):
        u = j % 4
        pltpu.make_async_copy(tab.at[idxs_all.at[j]], rows[u],
                              gsems[u]).wait()
        pltpu.async_copy(rows[u], acc_sh.at[idxd_all.at[j]], ssems[u],
                         add=True)
        pltpu.make_async_copy(rows[(u - 1) % 4], acc_sh.at[idxd_all.at[0]],
                              ssems[(u - 1) % 4]).wait()
    last = (NITER - 1) % 4
    pltpu.make_async_copy(rows[last], acc_sh.at[idxd_all.at[0]],
                          ssems[last]).wait()


def _deg_kernel(ei_hbm, dego_hbm, degi_hbm, idxs_all, idxd_all,
                ones_v, zbuf, dego_sh, degi_sh, isem, ssem):
    c = lax.axis_index("c")
    s = lax.axis_index("s")
    wid = c * NS + s

    cps = pltpu.async_copy(ei_hbm.at[0, wid], idxs_all, isem)
    cpd = pltpu.async_copy(ei_hbm.at[1, wid], idxd_all, isem)

    def fill(i, carry):
        ones_v[pl.ds(i * 16, 16)] = jnp.ones((16,), jnp.float32)
        return carry

    lax.fori_loop(0, 7, fill, 0)

    def zfill(i, carry):
        zbuf[pl.ds(i * 16, 16)] = jnp.zeros((16,), jnp.float32)
        return carry

    lax.fori_loop(0, RPT // 16, zfill, 0)
    pltpu.sync_copy(zbuf, dego_sh.at[pl.ds(s * RPT, RPT)])
    pltpu.sync_copy(zbuf, degi_sh.at[pl.ds(s * RPT, RPT)])
    cps.wait()
    cpd.wait()
    plsc.subcore_barrier()

    ones_c = ones_v.at[pl.ds(0, CHUNK)]

    def body(j, carry):
        @pl.when(j >= LAG)
        def _drain():
            pltpu.make_async_copy(ones_c, dego_sh.at[idxs_all.at[0]],
                                  ssem).wait()
            pltpu.make_async_copy(ones_c, degi_sh.at[idxd_all.at[0]],
                                  ssem).wait()

        pltpu.async_copy(ones_c, dego_sh.at[idxs_all.at[j]], ssem, add=True)
        pltpu.async_copy(ones_c, degi_sh.at[idxd_all.at[j]], ssem, add=True)
        return carry

    lax.fori_loop(0, NITER, body, 0)

    def drain(j, carry):
        pltpu.make_async_copy(ones_c, dego_sh.at[idxs_all.at[0]], ssem).wait()
        pltpu.make_async_copy(ones_c, degi_sh.at[idxd_all.at[0]], ssem).wait()
        return carry

    lax.fori_loop(0, LAG, drain, 0)
    plsc.subcore_barrier()

    pltpu.sync_copy(dego_sh.at[pl.ds(s * RPT, RPT)],
                    dego_hbm.at[c, 0, pl.ds(s * RPT, RPT)])
    pltpu.sync_copy(degi_sh.at[pl.ds(s * RPT, RPT)],
                    degi_hbm.at[c, 0, pl.ds(s * RPT, RPT)])


_deg_call = pl.kernel(
    _deg_kernel,
    out_type=[jax.ShapeDtypeStruct((NC, 1, NPAD), jnp.float32),
              jax.ShapeDtypeStruct((NC, 1, NPAD), jnp.float32)],
    mesh=_SC_MESH,
    compiler_params=_SC_PARAMS,
    scratch_types=[
        pltpu.VMEM((NITER, CHUNK), jnp.int32),
        pltpu.VMEM((NITER, CHUNK), jnp.int32),
        pltpu.VMEM((112,), jnp.float32),
        pltpu.VMEM((RPT,), jnp.float32),
        pltpu.VMEM_SHARED((NPAD,), jnp.float32),
        pltpu.VMEM_SHARED((NPAD,), jnp.float32),
        pltpu.SemaphoreType.DMA,
        pltpu.SemaphoreType.DMA,
    ],
)


def _edge_kernel(F, scale, *args):
    if scale:
        (ei_hbm, tab_hbm, norm_hbm, acc_hbm,
         idxs_all, idxd_all, r0, r1, r2, r3, zbuf, norm_v, acc_sh, tab_sh,
         isem, g0, g1, g2, g3, s0, s1, s2, s3) = args
    else:
        (ei_hbm, tab_hbm, acc_hbm,
         idxs_all, idxd_all, r0, r1, r2, r3, zbuf, acc_sh, tab_sh,
         isem, g0, g1, g2, g3, s0, s1, s2, s3) = args
    c = lax.axis_index("c")
    s = lax.axis_index("s")
    wid = c * NS + s
    rows = [r0, r1, r2, r3]
    gsems = [g0, g1, g2, g3]
    ssems = [s0, s1, s2, s3]

    cps = pltpu.async_copy(ei_hbm.at[0, wid], idxs_all, isem)
    cpd = pltpu.async_copy(ei_hbm.at[1, wid], idxd_all, isem)
    if scale:
        cpn = pltpu.async_copy(norm_hbm.at[pl.ds(s * RPT, RPT)], norm_v,
                               isem)
    else:
        cpt = pltpu.async_copy(tab_hbm.at[pl.ds(s * RPT, RPT)],
                               tab_sh.at[pl.ds(s * RPT, RPT)], isem)

    zrows = 32
    z16 = jnp.zeros((16,), jnp.float32)

    def zero_row(i, carry):
        for jj in range(F // 16):
            zbuf[i, pl.ds(jj * 16, 16)] = z16
        return carry

    lax.fori_loop(0, zrows, zero_row, 0)

    def zero_slice(k, carry):
        pltpu.sync_copy(zbuf, acc_sh.at[pl.ds(s * RPT + k * zrows, zrows)])
        return carry

    lax.fori_loop(0, RPT // zrows, zero_slice, 0)
    cps.wait()
    cpd.wait()
    if scale:
        cpn.wait()
        for k in range(RPT // CHUNK):
            b = k % 2
            pltpu.sync_copy(tab_hbm.at[pl.ds(s * RPT + k * CHUNK, CHUNK)],
                            rows[b])

            def scale_row(r, carry, _k=k, _b=b):
                nb = plsc.load_gather(
                    norm_v, [jnp.full((16,), _k * CHUNK + r, jnp.int32)])
                for jj in range(F // 16):
                    rows[_b][r, pl.ds(jj * 16, 16)] = (
                        rows[_b][r, pl.ds(jj * 16, 16)] * nb)
                return carry

            lax.fori_loop(0, CHUNK, scale_row, 0)
            pltpu.sync_copy(rows[b],
                            tab_sh.at[pl.ds(s * RPT + k * CHUNK, CHUNK)])
    else:
        cpt.wait()
    plsc.subcore_barrier()

    _edge_ring(tab_sh, acc_sh, idxs_all, idxd_all, rows, gsems, ssems)
    plsc.subcore_barrier()

    pltpu.sync_copy(acc_sh.at[pl.ds(s * RPT, RPT)],
                    acc_hbm.at[c, pl.ds(s * RPT, RPT)])


def _make_edge_call(F, scale):
    scratch = [
        pltpu.VMEM((NITER, CHUNK), jnp.int32),
        pltpu.VMEM((NITER, CHUNK), jnp.int32),
        pltpu.VMEM((CHUNK, F), jnp.float32),
        pltpu.VMEM((CHUNK, F), jnp.float32),
        pltpu.VMEM((CHUNK, F), jnp.float32),
        pltpu.VMEM((CHUNK, F), jnp.float32),
        pltpu.VMEM((32, F), jnp.float32),
    ]
    if scale:
        scratch.append(pltpu.VMEM((RPT,), jnp.float32))
    scratch += [
        pltpu.VMEM_SHARED((NPAD, F), jnp.float32),
        pltpu.VMEM_SHARED((NPAD, F), jnp.float32),
    ] + [pltpu.SemaphoreType.DMA] * 9
    return pl.kernel(
        functools.partial(_edge_kernel, F, scale),
        out_type=jax.ShapeDtypeStruct((NC, NPAD, F), jnp.float32),
        mesh=_SC_MESH,
        compiler_params=_SC_PARAMS,
        scratch_types=scratch,
    )


_edge_call_h = _make_edge_call(NH, True)
_edge_call_c = _make_edge_call(NCLS, False)


def _mm_body(x_ref, w_ref, o_ref):
    z = jnp.dot(x_ref[...], w_ref[...], preferred_element_type=jnp.float32)
    o_ref[0:NN, :] = z
    o_ref[NN:NPAD, :] = jnp.zeros((NPAD - NN, NH), jnp.float32)


_mm = pl.pallas_call(
    _mm_body,
    out_shape=jax.ShapeDtypeStruct((NPAD, NH), jnp.float32),
)


def _mid_body(acc_ref, ni_ref, no_ref, b1_ref, w2_ref, o_ref):
    h = acc_ref[0] + acc_ref[1]
    h = jnp.maximum(h * ni_ref[...] + b1_ref[...], 0.0)
    o_ref[...] = jnp.dot(h * no_ref[...], w2_ref[...],
                         preferred_element_type=jnp.float32)


_mid = pl.pallas_call(
    _mid_body,
    out_shape=jax.ShapeDtypeStruct((NPAD, NCLS), jnp.float32),
)


def _fin_body(acc_ref, ni_ref, b2_ref, o_ref):
    o_ref[...] = ((acc_ref[0, :NN, :] + acc_ref[1, :NN, :])
                  * ni_ref[0:NN] + b2_ref[...])


_fin = pl.pallas_call(
    _fin_body,
    out_shape=jax.ShapeDtypeStruct((NN, NCLS), jnp.float32),
)


@jax.jit
def kernel(features, edge_index, W1, b1, W2, b2):
    ei = edge_index.reshape(2, NW, NITER, CHUNK)

    dego, degi = _deg_call(ei)
    deg_out = dego[0, 0] + dego[1, 0]
    deg_in = degi[0, 0] + degi[1, 0]
    norm_out_f = lax.rsqrt(jnp.maximum(deg_out, 1.0))
    norm_out = norm_out_f.reshape(NPAD, 1)
    norm_in = lax.rsqrt(jnp.maximum(deg_in, 1.0)).reshape(NPAD, 1)

    z1 = _mm(features, W1)
    acc1 = _edge_call_h(ei, z1, norm_out_f)
    hw2 = _mid(acc1, norm_in, norm_out, b1.reshape(1, NH), W2)
    acc2 = _edge_call_c(ei, hw2)
    return _fin(acc2, norm_in, b2.reshape(1, NCLS))
